# baseline (speedup 1.0000x reference)
import functools

import jax
import jax.numpy as jnp
from jax import lax
from jax.experimental import pallas as pl
from jax.experimental.pallas import tpu as pltpu
from jax.experimental.pallas import tpu_sc as plsc

_G = 8
_NBUF = 3


@functools.partial(jax.jit, static_argnames=("n",))
def _broadcast_sc(pos_embed, n):
    seq_len, d_model = pos_embed.shape
    info = plsc.get_sparse_core_info()
    num_workers = info.num_cores * info.num_subcores
    assert seq_len % (num_workers * _G) == 0
    rows = seq_len // num_workers
    ngroups = rows // _G

    mesh = plsc.VectorSubcoreMesh(core_axis_name="c", subcore_axis_name="s")

    @functools.partial(
        pl.kernel,
        mesh=mesh,
        out_type=jax.ShapeDtypeStruct((seq_len, n, d_model), jnp.float32),
        scratch_types=[pltpu.VMEM((_NBUF, _G, n, d_model), jnp.float32)]
        + [pltpu.SemaphoreType.DMA] * (_NBUF + 1),
    )
    def k(emb_hbm, out_hbm, repl, *sems):
        rsems, wsem = sems[:_NBUF], sems[_NBUF]
        wid = lax.axis_index("s") * info.num_cores + lax.axis_index("c")
        base = wid * rows
        writes = []
        for g in range(ngroups):
            b = g % _NBUF
            if g >= _NBUF:
                writes[g - _NBUF].wait()
            reads = [
                pltpu.async_copy(
                    emb_hbm.at[pl.ds(base + g * _G, _G)],
                    repl.at[b, :, j],
                    rsems[b],
                )
                for j in range(n)
            ]
            for r in reads:
                r.wait()
            writes.append(
                pltpu.async_copy(
                    repl.at[b], out_hbm.at[pl.ds(base + g * _G, _G)], wsem
                )
            )
        for w in writes[-_NBUF:]:
            w.wait()

    return k(pos_embed)


def kernel(z, pos_embed):
    if z.ndim == 2:
        n = z.shape[0]
    elif z.ndim == 3:
        n = z.shape[1]
    else:
        raise Exception
    return _broadcast_sc(pos_embed, n)


# speedup vs baseline: 1.5747x; 1.5747x over previous
import functools

import jax
import jax.numpy as jnp
from jax import lax
from jax.experimental import pallas as pl
from jax.experimental.pallas import tpu as pltpu
from jax.experimental.pallas import tpu_sc as plsc


@functools.partial(jax.jit, static_argnames=("n",))
def _broadcast_sc(pos_embed, n):
    seq_len, d_model = pos_embed.shape
    info = plsc.get_sparse_core_info()
    num_workers = info.num_cores * info.num_subcores
    assert seq_len % num_workers == 0
    rows = seq_len // num_workers

    mesh = plsc.VectorSubcoreMesh(core_axis_name="c", subcore_axis_name="s")

    @functools.partial(
        pl.kernel,
        mesh=mesh,
        out_type=jax.ShapeDtypeStruct((seq_len, n, d_model), jnp.float32),
        scratch_types=[
            pltpu.VMEM((rows, d_model), jnp.float32),
            pltpu.SemaphoreType.DMA,
        ],
    )
    def k(emb_hbm, out_hbm, buf, sem):
        wid = lax.axis_index("s") * info.num_cores + lax.axis_index("c")
        base = wid * rows
        pltpu.sync_copy(emb_hbm.at[pl.ds(base, rows)], buf)
        copies = [
            pltpu.async_copy(buf, out_hbm.at[pl.ds(base, rows), j], sem)
            for j in range(n)
        ]
        for c in copies:
            c.wait()

    return k(pos_embed)


def kernel(z, pos_embed):
    if z.ndim == 2:
        n = z.shape[0]
    elif z.ndim == 3:
        n = z.shape[1]
    else:
        raise Exception
    return _broadcast_sc(pos_embed, n)


# speedup vs baseline: 1.5862x; 1.0073x over previous
import functools

import jax
import jax.numpy as jnp
from jax import lax
from jax.experimental import pallas as pl
from jax.experimental.pallas import tpu as pltpu
from jax.experimental.pallas import tpu_sc as plsc


@functools.partial(jax.jit, static_argnames=("n",))
def _broadcast_sc(pos_embed, n):
    seq_len, d_model = pos_embed.shape
    info = plsc.get_sparse_core_info()
    num_workers = info.num_cores * info.num_subcores
    assert seq_len % num_workers == 0
    rows = seq_len // num_workers

    mesh = plsc.VectorSubcoreMesh(core_axis_name="c", subcore_axis_name="s")

    @functools.partial(
        pl.kernel,
        mesh=mesh,
        out_type=jax.ShapeDtypeStruct((seq_len, n, d_model), jnp.float32),
        scratch_types=[
            pltpu.VMEM((rows, 1, d_model), jnp.float32),
            pltpu.SemaphoreType.DMA,
        ],
    )
    def k(emb_hbm, out_hbm, buf, sem):
        wid = lax.axis_index("s") * info.num_cores + lax.axis_index("c")
        base = wid * rows
        pltpu.sync_copy(emb_hbm.at[pl.ds(base, rows)], buf.at[:, 0])
        copies = [
            pltpu.async_copy(
                buf, out_hbm.at[pl.ds(base, rows), pl.ds(j, 1)], sem
            )
            for j in range(n)
        ]
        for c in copies:
            c.wait()

    return k(pos_embed)


def kernel(z, pos_embed):
    if z.ndim == 2:
        n = z.shape[0]
    elif z.ndim == 3:
        n = z.shape[1]
    else:
        raise Exception
    return _broadcast_sc(pos_embed, n)
